# Initial kernel scaffold; baseline (speedup 1.0000x reference)
import jax
import jax.numpy as jnp
from jax.experimental import pallas as pl


def kernel(x, edge_index, edge_attr, edge_weight, Wb, bb, We0, be0, We1, be1, We2, be2, linW, linb, centerW, Wt0, bt0, gamma, beta, Wt1, bt1):
    raise NotImplementedError("write your pallas kernel here")


# speedup vs baseline: 1.9981x; 1.9981x over previous
import functools

import jax
import jax.numpy as jnp
from jax import lax
from jax.experimental import pallas as pl
from jax.experimental.pallas import tpu as pltpu
from jax.experimental.pallas import tpu_sc as plsc

N = 10000
E = 320000
H = 128
BD = 16
EPS = 1e-5
LN2 = 0.6931471805599453

NC = 2
NS = 16
L = 16
NW = NC * NS
EPW = E // NW
CH = 80
NCHUNK = EPW // CH
NRCH = N // CH


def _ceil_div(a, b):
    return -(-a // b)

NB = 2000
EB = 2560


def _ssp(v):
    return jnp.maximum(v, 0.0) + jnp.log(1.0 + jnp.exp(-jnp.abs(v))) - LN2


def _nodeproj_body(x_ref, ws_ref, wd_ref, wl_ref, lb_ref, wb_ref, bb_ref,
                   w0b_ref, be0_ref, as_ref, ad_ref, xl_ref, wc_ref, bc_ref):
    x = x_ref[...]
    as_ref[...] = jnp.dot(x, ws_ref[...], preferred_element_type=jnp.float32)
    ad_ref[...] = jnp.dot(x, wd_ref[...], preferred_element_type=jnp.float32)
    xl_ref[...] = jnp.dot(x, wl_ref[...], preferred_element_type=jnp.float32) + lb_ref[...]

    @pl.when(pl.program_id(0) == 0)
    def _():
        w0b = w0b_ref[...]
        wc_ref[...] = jnp.dot(wb_ref[...], w0b, preferred_element_type=jnp.float32)
        bc_ref[...] = jnp.dot(bb_ref[...], w0b, preferred_element_type=jnp.float32) + be0_ref[...]


def _node_proj(x, We0_s, We0_d, linW, linb2, Wb, bb2, We0_b, be02):
    grid = (N // NB,)
    full = lambda i: (0, 0)
    return pl.pallas_call(
        _nodeproj_body,
        grid=grid,
        in_specs=[
            pl.BlockSpec((NB, H), lambda i: (i, 0)),
            pl.BlockSpec((H, H), full),
            pl.BlockSpec((H, H), full),
            pl.BlockSpec((H, H), full),
            pl.BlockSpec((1, H), full),
            pl.BlockSpec((BD, H), full),
            pl.BlockSpec((1, H), full),
            pl.BlockSpec((H, H), full),
            pl.BlockSpec((1, H), full),
        ],
        out_specs=[
            pl.BlockSpec((NB, H), lambda i: (i, 0)),
            pl.BlockSpec((NB, H), lambda i: (i, 0)),
            pl.BlockSpec((NB, H), lambda i: (i, 0)),
            pl.BlockSpec((BD, H), full),
            pl.BlockSpec((1, H), full),
        ],
        out_shape=[
            jax.ShapeDtypeStruct((N, H), jnp.float32),
            jax.ShapeDtypeStruct((N, H), jnp.float32),
            jax.ShapeDtypeStruct((N, H), jnp.float32),
            jax.ShapeDtypeStruct((BD, H), jnp.float32),
            jax.ShapeDtypeStruct((1, H), jnp.float32),
        ],
    )(x, We0_s, We0_d, linW, linb2, Wb, bb2, We0_b, be02)


def _sc_gather_body(as_hbm, ad_hbm, src_hbm, dst_hbm, gsum_hbm,
                    idx_s, idx_d, rows_s, rows_d, sem1, sem2):
    wid = lax.axis_index("s") * NC + lax.axis_index("c")
    base0 = wid * EPW

    def chunk(i, carry):
        base = base0 + i * CH
        pltpu.sync_copy(src_hbm.at[pl.ds(base, CH)], idx_s)
        pltpu.sync_copy(dst_hbm.at[pl.ds(base, CH)], idx_d)
        c1 = pltpu.async_copy(as_hbm.at[idx_s], rows_s, sem1)
        c2 = pltpu.async_copy(ad_hbm.at[idx_d], rows_d, sem2)
        c1.wait()
        c2.wait()

        def row(r, c):
            for g in range(H // L):
                sl = pl.ds(g * L, L)
                rows_s[r, sl] = rows_s[r, sl] + rows_d[r, sl]
            return c

        lax.fori_loop(0, CH, row, 0, unroll=4)
        pltpu.sync_copy(rows_s, gsum_hbm.at[pl.ds(base, CH)])
        return carry

    lax.fori_loop(0, NCHUNK, chunk, 0)


@functools.cache
def _sc_gather_kernel():
    return pl.kernel(
        _sc_gather_body,
        out_type=jax.ShapeDtypeStruct((E, H), jnp.float32),
        mesh=plsc.VectorSubcoreMesh(core_axis_name="c", subcore_axis_name="s",
                                    num_cores=NC, num_subcores=NS),
        scratch_types=[
            pltpu.VMEM((CH,), jnp.int32),
            pltpu.VMEM((CH,), jnp.int32),
            pltpu.VMEM((CH, H), jnp.float32),
            pltpu.VMEM((CH, H), jnp.float32),
            pltpu.SemaphoreType.DMA,
            pltpu.SemaphoreType.DMA,
        ],
    )


def _sc_gather(As, Ad, src, dst):
    return _sc_gather_kernel()(As, Ad, src, dst)


def _edgemlp_body(ea_ref, gs_ref, ew_ref, wc_ref, bc_ref, w1_ref, b1_ref,
                  w2_ref, b2_ref, hw_ref):
    h0 = jnp.dot(ea_ref[...], wc_ref[...], preferred_element_type=jnp.float32)
    h0 = h0 + gs_ref[...] + bc_ref[...]
    h = _ssp(h0)
    h = _ssp(jnp.dot(h, w1_ref[...], preferred_element_type=jnp.float32) + b1_ref[...])
    h2 = jnp.dot(h, w2_ref[...], preferred_element_type=jnp.float32) + b2_ref[...]
    hw_ref[...] = h2 * ew_ref[...]


def _edge_mlp(edge_attr, gsum, ew2, Wcomb, bcomb, We1, be12, We2, be22):
    grid = (E // EB,)
    full = lambda i: (0, 0)
    return pl.pallas_call(
        _edgemlp_body,
        grid=grid,
        in_specs=[
            pl.BlockSpec((EB, BD), lambda i: (i, 0)),
            pl.BlockSpec((EB, H), lambda i: (i, 0)),
            pl.BlockSpec((EB, 1), lambda i: (i, 0)),
            pl.BlockSpec((BD, H), full),
            pl.BlockSpec((1, H), full),
            pl.BlockSpec((H, H), full),
            pl.BlockSpec((1, H), full),
            pl.BlockSpec((H, H), full),
            pl.BlockSpec((1, H), full),
        ],
        out_specs=pl.BlockSpec((EB, H), lambda i: (i, 0)),
        out_shape=jax.ShapeDtypeStruct((E, H), jnp.float32),
    )(edge_attr, gsum, ew2, Wcomb, bcomb, We1, be12, We2, be22)


def _sc_scatter_body(xl_hbm, hw_hbm, src_hbm, dst_hbm, out_hbm,
                     agg, idx_s, idx_d, rows_xl, rows_hw, zbuf, sem1):
    cid = lax.axis_index("c")
    sid = lax.axis_index("s")
    wid = sid * NC + cid
    base0 = wid * EPW

    zero = jnp.zeros((L,), jnp.float32)

    def zrow(r, c):
        for g in range(H // L):
            zbuf[r, pl.ds(g * L, L)] = zero
        return c

    lax.fori_loop(0, CH, zrow, 0, unroll=4)
    for k in range(_ceil_div(NRCH, NS)):
        cidx = sid + k * NS

        @pl.when(cidx < NRCH)
        def _():
            pltpu.sync_copy(zbuf, agg.at[pl.ds(cidx * CH, CH)])

    plsc.subcore_barrier()

    def chunk(i, carry):
        base = base0 + i * CH
        pltpu.sync_copy(src_hbm.at[pl.ds(base, CH)], idx_s)
        pltpu.sync_copy(dst_hbm.at[pl.ds(base, CH)], idx_d)
        c1 = pltpu.async_copy(xl_hbm.at[idx_s], rows_xl, sem1)
        pltpu.sync_copy(hw_hbm.at[pl.ds(base, CH)], rows_hw)
        c1.wait()

        def row(r, c):
            for g in range(H // L):
                sl = pl.ds(g * L, L)
                rows_hw[r, sl] = rows_hw[r, sl] * rows_xl[r, sl]
            return c

        lax.fori_loop(0, CH, row, 0, unroll=4)
        pltpu.sync_copy(rows_hw, agg.at[idx_d], add=True)
        return carry

    lax.fori_loop(0, NCHUNK, chunk, 0)
    plsc.subcore_barrier()

    for k in range(_ceil_div(NRCH, NS)):
        cidx = sid + k * NS

        @pl.when(cidx < NRCH)
        def _():
            pltpu.sync_copy(agg.at[pl.ds(cidx * CH, CH)],
                            out_hbm.at[cid, pl.ds(cidx * CH, CH)])


@functools.cache
def _sc_scatter_kernel():
    return pl.kernel(
        _sc_scatter_body,
        out_type=jax.ShapeDtypeStruct((NC, N, H), jnp.float32),
        mesh=plsc.VectorSubcoreMesh(core_axis_name="c", subcore_axis_name="s",
                                    num_cores=NC, num_subcores=NS),
        scratch_types=[
            pltpu.VMEM_SHARED((N, H), jnp.float32),
            pltpu.VMEM((CH,), jnp.int32),
            pltpu.VMEM((CH,), jnp.int32),
            pltpu.VMEM((CH, H), jnp.float32),
            pltpu.VMEM((CH, H), jnp.float32),
            pltpu.VMEM((CH, H), jnp.float32),
            pltpu.SemaphoreType.DMA,
        ],
    )


def _sc_scatter(xl, hw, src, dst):
    return _sc_scatter_kernel()(xl, hw, src, dst)


def _trans1_body(p_ref, xl_ref, cw_ref, wt0_ref, bt0_ref, t_ref, s_ref, ss_ref):
    xo = p_ref[0] + p_ref[1] + cw_ref[...] * xl_ref[...]
    t = jnp.dot(xo, wt0_ref[...], preferred_element_type=jnp.float32) + bt0_ref[...]
    t_ref[...] = t

    @pl.when(pl.program_id(0) == 0)
    def _():
        s_ref[...] = jnp.zeros_like(s_ref)
        ss_ref[...] = jnp.zeros_like(ss_ref)

    s_ref[...] += jnp.sum(t, axis=0, keepdims=True)
    ss_ref[...] += jnp.sum(t * t, axis=0, keepdims=True)


def _trans1(partials, xl, centerW, Wt0, bt02):
    grid = (N // NB,)
    full = lambda i: (0, 0)
    return pl.pallas_call(
        _trans1_body,
        grid=grid,
        in_specs=[
            pl.BlockSpec((NC, NB, H), lambda i: (0, i, 0)),
            pl.BlockSpec((NB, H), lambda i: (i, 0)),
            pl.BlockSpec((1, H), full),
            pl.BlockSpec((H, H), full),
            pl.BlockSpec((1, H), full),
        ],
        out_specs=[
            pl.BlockSpec((NB, H), lambda i: (i, 0)),
            pl.BlockSpec((1, H), full),
            pl.BlockSpec((1, H), full),
        ],
        out_shape=[
            jax.ShapeDtypeStruct((N, H), jnp.float32),
            jax.ShapeDtypeStruct((1, H), jnp.float32),
            jax.ShapeDtypeStruct((1, H), jnp.float32),
        ],
    )(partials, xl, centerW, Wt0, bt02)


def _trans2_body(t_ref, s_ref, ss_ref, g_ref, b_ref, wt1_ref, bt1_ref, o_ref):
    mu = s_ref[...] * (1.0 / N)
    var = ss_ref[...] * (1.0 / N) - mu * mu
    inv = lax.rsqrt(var + EPS) * g_ref[...]
    th = (t_ref[...] - mu) * inv + b_ref[...]
    o_ref[...] = jnp.dot(_ssp(th), wt1_ref[...], preferred_element_type=jnp.float32) + bt1_ref[...]


def _trans2(t, s, ss, gamma2, beta2, Wt1, bt12):
    grid = (N // NB,)
    full = lambda i: (0, 0)
    return pl.pallas_call(
        _trans2_body,
        grid=grid,
        in_specs=[
            pl.BlockSpec((NB, H), lambda i: (i, 0)),
            pl.BlockSpec((1, H), full),
            pl.BlockSpec((1, H), full),
            pl.BlockSpec((1, H), full),
            pl.BlockSpec((1, H), full),
            pl.BlockSpec((H, H), full),
            pl.BlockSpec((1, H), full),
        ],
        out_specs=pl.BlockSpec((NB, H), lambda i: (i, 0)),
        out_shape=jax.ShapeDtypeStruct((N, H), jnp.float32),
    )(t, s, ss, gamma2, beta2, Wt1, bt12)


def kernel(x, edge_index, edge_attr, edge_weight, Wb, bb, We0, be0, We1, be1,
           We2, be2, linW, linb, centerW, Wt0, bt0, gamma, beta, Wt1, bt1):
    We0_b = We0[:H]
    We0_s = We0[H:2 * H]
    We0_d = We0[2 * H:]
    r1 = lambda v: v.reshape(1, H)

    As, Ad, xl, Wcomb, bcomb = _node_proj(
        x, We0_s, We0_d, linW, r1(linb), Wb, r1(bb), We0_b, r1(be0))

    src = edge_index[0]
    dst = edge_index[1]
    gsum = _sc_gather(As, Ad, src, dst)

    hw = _edge_mlp(edge_attr, gsum, edge_weight.reshape(E, 1),
                   Wcomb, bcomb, We1, r1(be1), We2, r1(be2))

    partials = _sc_scatter(xl, hw, src, dst)

    t, s, ss = _trans1(partials, xl, centerW, Wt0, r1(bt0))
    out = _trans2(t, s, ss, r1(gamma), r1(beta), Wt1, r1(bt1))
    return out
